# Initial kernel scaffold; baseline (speedup 1.0000x reference)
import jax
import jax.numpy as jnp
from jax.experimental import pallas as pl


def kernel(node_feat, edge_index, edge_attr, W1, b1, We1, be1, W2, b2, We2, be2, g1, bt1, g2, bt2):
    raise NotImplementedError("write your pallas kernel here")


# speedup vs baseline: 1.4494x; 1.4494x over previous
import functools

import jax
import jax.numpy as jnp
from jax import lax
from jax.experimental import pallas as pl
from jax.experimental.pallas import tpu as pltpu
from jax.experimental.pallas import tpu_sc as plsc

N = 10000
E = 160000
S = 10
D_NODE = 256
D_HID = 256
N_CLASSES = 128

NC = 2
NS = 16
NW = NC * NS

NPW = 320
NPAD = NW * NPW
CH = 8
NIDX = CH * S
NCHUNK = NPW // CH


def _sc_gather_sums(h_tab, ea_tab, eid_pad, gidx_pad):
    mesh = plsc.VectorSubcoreMesh(core_axis_name="c", subcore_axis_name="s")

    @functools.partial(
        pl.kernel,
        out_type=jax.ShapeDtypeStruct((NPAD, 384), jnp.float32),
        mesh=mesh,
        scratch_types=[
            pltpu.VMEM((NIDX,), jnp.int32),
            pltpu.VMEM((NIDX,), jnp.int32),
            pltpu.VMEM((NIDX, 128), jnp.float32),
            pltpu.VMEM((NIDX, D_NODE), jnp.float32),
            pltpu.VMEM((CH, 384), jnp.float32),
            pltpu.SemaphoreType.DMA,
            pltpu.SemaphoreType.DMA,
        ],
    )
    def k(h_hbm, ea_hbm, eid_hbm, gidx_hbm, out_hbm,
          eid_v, gidx_v, ea_buf, h_buf, stage, sem1, sem2):
        wid = lax.axis_index("s") * NC + lax.axis_index("c")

        def chunk_body(j, carry):
            base_i = pl.multiple_of(wid * (NPW * S) + j * NIDX, 8)
            node0 = pl.multiple_of(wid * NPW + j * CH, 8)
            pltpu.sync_copy(eid_hbm.at[pl.ds(base_i, NIDX)], eid_v)
            pltpu.sync_copy(gidx_hbm.at[pl.ds(base_i, NIDX)], gidx_v)
            cp1 = pltpu.async_copy(ea_hbm.at[eid_v], ea_buf, sem1)
            cp2 = pltpu.async_copy(h_hbm.at[gidx_v], h_buf, sem2)
            cp1.wait()
            cp2.wait()
            for n in range(CH):
                ea_acc = ea_buf[n * S, pl.ds(0, 16)]
                for s in range(1, S):
                    ea_acc = ea_acc + ea_buf[n * S + s, pl.ds(0, 16)]
                stage[n, pl.ds(D_NODE, 16)] = ea_acc
                for kk in range(D_NODE // 16):
                    acc = h_buf[n * S, pl.ds(kk * 16, 16)]
                    for s in range(1, S):
                        acc = acc + h_buf[n * S + s, pl.ds(kk * 16, 16)]
                    stage[n, pl.ds(kk * 16, 16)] = acc
            pltpu.sync_copy(stage, out_hbm.at[pl.ds(node0, CH)])
            return carry

        lax.fori_loop(0, NCHUNK, chunk_body, 0)

    return k(h_tab, ea_tab, eid_pad, gidx_pad)


def _tc_dense(h, sums, maskf, We, be, W, b, g, bt, d_in, d_out):
    BLK = 1024

    def body(h_ref, sums_ref, m_ref, We_ref, be_ref, W_ref, b_ref,
             g_ref, bt_ref, o_ref):
        inv_s = jnp.float32(1.0 / S)
        hs = sums_ref[:, 0:D_NODE]
        ea = sums_ref[:, D_NODE:D_NODE + 16]
        neigh = (hs * inv_s
                 + jnp.dot(ea * inv_s, We_ref[...],
                           preferred_element_type=jnp.float32)
                 + be_ref[...]) * m_ref[...]
        z = (jnp.dot(h_ref[...], W_ref[0:d_in, :],
                     preferred_element_type=jnp.float32)
             + jnp.dot(neigh, W_ref[d_in:2 * d_in, :],
                       preferred_element_type=jnp.float32)
             + b_ref[...])
        z = jnp.maximum(z, 0.0)
        mu = jnp.mean(z, axis=-1, keepdims=True)
        zc = z - mu
        var = jnp.mean(zc * zc, axis=-1, keepdims=True)
        y = zc * lax.rsqrt(var + 1e-5) * g_ref[...] + bt_ref[...]
        o_ref[...] = jnp.maximum(y, 0.0)

    return pl.pallas_call(
        body,
        grid=(NPAD // BLK,),
        in_specs=[
            pl.BlockSpec((BLK, d_in), lambda i: (i, 0)),
            pl.BlockSpec((BLK, 384), lambda i: (i, 0)),
            pl.BlockSpec((BLK, 1), lambda i: (i, 0)),
            pl.BlockSpec((16, d_in), lambda i: (0, 0)),
            pl.BlockSpec((1, d_in), lambda i: (0, 0)),
            pl.BlockSpec((2 * d_in, d_out), lambda i: (0, 0)),
            pl.BlockSpec((1, d_out), lambda i: (0, 0)),
            pl.BlockSpec((1, d_out), lambda i: (0, 0)),
            pl.BlockSpec((1, d_out), lambda i: (0, 0)),
        ],
        out_specs=pl.BlockSpec((BLK, d_out), lambda i: (i, 0)),
        out_shape=jax.ShapeDtypeStruct((NPAD, d_out), jnp.float32),
    )(h, sums, maskf, We, be.reshape(1, -1), W, b.reshape(1, -1),
      g.reshape(1, -1), bt.reshape(1, -1))


def kernel(node_feat, edge_index, edge_attr, W1, b1, We1, be1,
           W2, b2, We2, be2, g1, bt1, g2, bt2):
    src = edge_index[0]
    dst = edge_index[1]

    deg = jnp.bincount(dst, length=N)
    order = jnp.argsort(dst)
    starts = jnp.cumsum(deg) - deg
    maskf = (deg > 0).astype(jnp.float32)[:, None]
    maskf = jnp.pad(maskf, ((0, NPAD - N), (0, 0)))

    base_key = jax.random.key(42)
    eids = []
    for i in range(2):
        u = jax.random.uniform(jax.random.fold_in(base_key, i), (N, S))
        local = jnp.floor(u * jnp.maximum(deg, 1)[:, None]).astype(jnp.int32)
        pos = jnp.clip(starts[:, None] + local, 0, E - 1)
        eid = order[pos].reshape(-1).astype(jnp.int32)
        eids.append(jnp.pad(eid, (0, NPAD * S - N * S)))

    h = jnp.pad(node_feat, ((0, NPAD - N), (0, 0)))
    ea_pad = jnp.pad(edge_attr, ((0, 0), (0, 128 - 16)))
    layer_params = [(W1, b1, We1, be1, g1, bt1, D_NODE, D_HID),
                    (W2, b2, We2, be2, g2, bt2, D_HID, N_CLASSES)]
    for i, (W, b, We, be, gm, bt, d_in, d_out) in enumerate(layer_params):
        eid = eids[i]
        gidx = src[eid].astype(jnp.int32)
        sums = _sc_gather_sums(h, ea_pad, eid, gidx)
        h = _tc_dense(h, sums, maskf, We, be, W, b, gm, bt, d_in, d_out)
    return h[:N]
